# baseline (speedup 1.0000x reference)
import functools

import jax
import jax.numpy as jnp
from jax import lax
from jax.experimental import pallas as pl
from jax.experimental.pallas import tpu as pltpu
from jax.experimental.pallas import tpu_sc as plsc

_NUM_MOVIES = 100000
_D = 64
_NU = 512
_W32 = _NU // 4
_K = 200
_BPW = 8
_NWORK = _K // _BPW
_NS = 16


def _byte_total(s):
    return (
        (s & 0xFF)
        + (lax.shift_right_logical(s, 8) & 0xFF)
        + (lax.shift_right_logical(s, 16) & 0xFF)
        + (lax.shift_right_logical(s, 24) & 0xFF)
    )


def _sc_body(q_hbm, idx_hbm, tab_hbm, us_hbm, out_e_hbm, out_i_hbm,
             idx_v, qidx_v, rows_v, qrow_v, suset_v, quset_v, cat_v, iou_v,
             sem):
    wid = lax.axis_index("c") * _NS + lax.axis_index("s")

    @pl.when(wid < _NWORK)
    def _():
        base = wid * _BPW
        pltpu.sync_copy(idx_hbm.at[pl.ds(base, _BPW)], idx_v)
        pltpu.sync_copy(q_hbm, qidx_v)
        c1 = pltpu.make_async_copy(tab_hbm.at[idx_v], rows_v, sem)
        c2 = pltpu.make_async_copy(tab_hbm.at[qidx_v], qrow_v, sem)
        c3 = pltpu.make_async_copy(us_hbm.at[idx_v], suset_v, sem)
        c4 = pltpu.make_async_copy(us_hbm.at[qidx_v], quset_v, sem)
        c1.start()
        c2.start()
        c3.start()
        c4.start()
        c1.wait()
        c2.wait()
        c3.wait()
        c4.wait()

        qc = [qrow_v[0, pl.ds(c * 16, 16)] for c in range(_D // 16)]
        for j in range(_BPW):
            for c in range(_D // 16):
                cat_v[j, pl.ds(c * 16, 16)] = rows_v[j, pl.ds(c * 16, 16)]
                cat_v[j, pl.ds(_D + c * 16, 16)] = qc[c]

        qw = [quset_v[0, pl.ds(c * 16, 16)] for c in range(_W32 // 16)]
        qacc = qw[0]
        for c in range(1, _W32 // 16):
            qacc = qacc + qw[c]
        q_cnt = _byte_total(jnp.sum(qacc))

        lane = lax.iota(jnp.int32, 16)
        s_cnt_vec = jnp.zeros((16,), jnp.int32)
        i_cnt_vec = jnp.zeros((16,), jnp.int32)
        for j in range(_BPW):
            sw = suset_v[j, pl.ds(0, 16)]
            sacc = sw
            iacc = sw & qw[0]
            for c in range(1, _W32 // 16):
                sw = suset_v[j, pl.ds(c * 16, 16)]
                sacc = sacc + sw
                iacc = iacc + (sw & qw[c])
            s_cnt_vec = jnp.where(lane == j, _byte_total(jnp.sum(sacc)),
                                  s_cnt_vec)
            i_cnt_vec = jnp.where(lane == j, _byte_total(jnp.sum(iacc)),
                                  i_cnt_vec)
        s_len = s_cnt_vec.astype(jnp.float32)
        inter = i_cnt_vec.astype(jnp.float32)
        q_len = jnp.full((16,), 1.0, jnp.float32) * q_cnt.astype(jnp.float32)
        union = q_len + s_len - inter
        iou_v[...] = jnp.where(
            union > 0, inter / jnp.maximum(union, 1.0), 0.0)

        pltpu.sync_copy(cat_v, out_e_hbm.at[pl.ds(base, _BPW)])
        pltpu.sync_copy(iou_v.at[pl.ds(0, _BPW)],
                        out_i_hbm.at[pl.ds(base, _BPW)])


@functools.lru_cache(maxsize=None)
def _build_sc_kernel():
    return pl.kernel(
        _sc_body,
        out_type=(
            jax.ShapeDtypeStruct((_K, 2 * _D), jnp.float32),
            jax.ShapeDtypeStruct((_K,), jnp.float32),
        ),
        mesh=plsc.VectorSubcoreMesh(core_axis_name="c", subcore_axis_name="s"),
        compiler_params=pltpu.CompilerParams(
            needs_layout_passes=False, use_tc_tiling_on_sc=False),
        scratch_types=[
            pltpu.VMEM((_BPW,), jnp.int32),
            pltpu.VMEM((1,), jnp.int32),
            pltpu.VMEM((_BPW, _D), jnp.float32),
            pltpu.VMEM((1, _D), jnp.float32),
            pltpu.VMEM((_BPW, _W32), jnp.int32),
            pltpu.VMEM((1, _W32), jnp.int32),
            pltpu.VMEM((_BPW, 2 * _D), jnp.float32),
            pltpu.VMEM((16,), jnp.float32),
            pltpu.SemaphoreType.DMA,
        ],
    )


def kernel(query, support_set, embed_table, user_sets):
    u32 = jax.lax.bitcast_convert_type(
        user_sets.view(jnp.uint8).reshape(_NUM_MOVIES, _W32, 4), jnp.int32)
    cat_embeds, iou = _build_sc_kernel()(query, support_set, embed_table, u32)
    return cat_embeds, iou[:, None]


# speedup vs baseline: 6.2181x; 6.2181x over previous
import functools

import jax
import jax.numpy as jnp
from jax import lax
from jax.experimental import pallas as pl
from jax.experimental.pallas import tpu as pltpu
from jax.experimental.pallas import tpu_sc as plsc

_NUM_MOVIES = 100000
_D = 64
_NU = 512
_K = 200
_BPW = 8
_NWORK = _K // _BPW
_NS = 16


def _sc_body(q_hbm, idx_hbm, tabT_hbm, out_e_hbm,
             idx_v, slab_v, cat_v, sem):
    wid = lax.axis_index("c") * _NS + lax.axis_index("s")

    @pl.when(wid < _NWORK)
    def _():
        base = wid * _BPW
        pltpu.sync_copy(idx_hbm.at[pl.ds(base, _BPW)], idx_v.at[pl.ds(0, _BPW)])
        pltpu.sync_copy(q_hbm, idx_v.at[pl.ds(_BPW, 1)])
        ivec = idx_v[...]

        copies = []
        for j in range(_BPW + 1):
            m = ivec[j]
            start = pl.multiple_of((m // 128) * 128, 128)
            copies.append(pltpu.make_async_copy(
                tabT_hbm.at[:, pl.ds(start, 128)], slab_v.at[j], sem))
        for c in copies:
            c.start()
        for c in copies:
            c.wait()

        lane = lax.iota(jnp.int32, 16)
        offs = ivec % 128
        qoff = jnp.full((16,), offs[_BPW], jnp.int32)
        for c in range(_D // 16):
            qchunk = plsc.load_gather(
                slab_v, [jnp.full((16,), _BPW, jnp.int32), lane + c * 16,
                         qoff])
            for j in range(_BPW):
                joff = jnp.full((16,), offs[j], jnp.int32)
                cat_v[j, pl.ds(c * 16, 16)] = plsc.load_gather(
                    slab_v, [jnp.full((16,), j, jnp.int32), lane + c * 16,
                             joff])
                cat_v[j, pl.ds(_D + c * 16, 16)] = qchunk

        pltpu.sync_copy(cat_v, out_e_hbm.at[pl.ds(base, _BPW)])


@functools.lru_cache(maxsize=None)
def _build_sc_kernel():
    return pl.kernel(
        _sc_body,
        out_type=jax.ShapeDtypeStruct((_K, 2 * _D), jnp.float32),
        mesh=plsc.VectorSubcoreMesh(core_axis_name="c", subcore_axis_name="s"),
        compiler_params=pltpu.CompilerParams(needs_layout_passes=False),
        scratch_types=[
            pltpu.VMEM((16,), jnp.int32),
            pltpu.VMEM((_BPW + 1, _D, 128), jnp.float32),
            pltpu.VMEM((_BPW, 2 * _D), jnp.float32),
            pltpu.SemaphoreType.DMA,
        ],
    )


def _tc_iou_body(sidx_ref, qidx_ref, qblk_ref, *args):
    sblk_refs = args[:_BPW]
    out_ref = args[_BPW]
    g = pl.program_id(0)

    sub = lax.broadcasted_iota(jnp.int32, (32, _NU), 0)
    qrow_mask = (sub == qidx_ref[0] % 32).astype(jnp.float32)
    qblk = qblk_ref[...].astype(jnp.float32)
    qrow = jnp.sum(qblk * qrow_mask, axis=0, keepdims=True)
    q_len = jnp.sum(qrow)

    lane8 = lax.broadcasted_iota(jnp.int32, (1, _BPW), 1)
    iou_acc = jnp.zeros((1, _BPW), jnp.float32)
    for j in range(_BPW):
        m = sidx_ref[g * _BPW + j]
        rmask = (sub == m % 32).astype(jnp.float32)
        sblk = sblk_refs[j][...].astype(jnp.float32)
        srow_m = sblk * rmask
        s_len = jnp.sum(srow_m)
        inter = jnp.sum(srow_m * qrow)
        union = q_len + s_len - inter
        iou_j = jnp.where(union > 0, inter / jnp.maximum(union, 1.0), 0.0)
        iou_acc = jnp.where(lane8 == j, iou_j, iou_acc)
    out_ref[pl.ds(g, 1), :] = iou_acc


@functools.lru_cache(maxsize=None)
def _build_tc_kernel():
    def sblk_spec(j):
        return pl.BlockSpec(
            (32, _NU), lambda g, sidx, qidx, j=j: (sidx[g * _BPW + j] // 32, 0))

    return pl.pallas_call(
        _tc_iou_body,
        grid_spec=pltpu.PrefetchScalarGridSpec(
            num_scalar_prefetch=2,
            grid=(_K // _BPW,),
            in_specs=[
                pl.BlockSpec((32, _NU), lambda g, sidx, qidx: (qidx[0] // 32, 0)),
            ] + [sblk_spec(j) for j in range(_BPW)],
            out_specs=pl.BlockSpec((_K // _BPW, _BPW),
                                   lambda g, sidx, qidx: (0, 0)),
        ),
        out_shape=jax.ShapeDtypeStruct((_K // _BPW, _BPW), jnp.float32),
    )


def kernel(query, support_set, embed_table, user_sets):
    cat_embeds = _build_sc_kernel()(query, support_set, embed_table.T)
    iou = _build_tc_kernel()(
        support_set, query, *([user_sets] * (_BPW + 1)))
    return cat_embeds, iou.reshape(_K, 1)


# speedup vs baseline: 6.4615x; 1.0392x over previous
import functools

import jax
import jax.numpy as jnp
from jax import lax
from jax.experimental import pallas as pl
from jax.experimental.pallas import tpu as pltpu
from jax.experimental.pallas import tpu_sc as plsc

_NUM_MOVIES = 100000
_D = 64
_NU = 512
_K = 200
_BPW = 8
_NWORK = _K // _BPW
_NS = 16


def _sc_body(q_hbm, idx_hbm, tabT_hbm, out_e_hbm,
             idx_v, slab_v, cat_v, sem):
    wid = lax.axis_index("c") * _NS + lax.axis_index("s")

    @pl.when(wid < _NWORK)
    def _():
        base = wid * _BPW
        pltpu.sync_copy(idx_hbm.at[pl.ds(base, _BPW)], idx_v.at[pl.ds(0, _BPW)])
        pltpu.sync_copy(q_hbm, idx_v.at[pl.ds(_BPW, 1)])
        ivec = idx_v[...]

        copies = []
        for j in range(_BPW + 1):
            m = ivec[j]
            start = pl.multiple_of((m // 128) * 128, 128)
            copies.append(pltpu.make_async_copy(
                tabT_hbm.at[:, pl.ds(start, 128)], slab_v.at[j], sem))
        for c in copies:
            c.start()
        for c in copies:
            c.wait()

        lane = lax.iota(jnp.int32, 16)
        offs = ivec % 128
        qoff = jnp.full((16,), offs[_BPW], jnp.int32)
        for c in range(_D // 16):
            qchunk = plsc.load_gather(
                slab_v, [jnp.full((16,), _BPW, jnp.int32), lane + c * 16,
                         qoff])
            for j in range(_BPW):
                joff = jnp.full((16,), offs[j], jnp.int32)
                cat_v[j, pl.ds(c * 16, 16)] = plsc.load_gather(
                    slab_v, [jnp.full((16,), j, jnp.int32), lane + c * 16,
                             joff])
                cat_v[j, pl.ds(_D + c * 16, 16)] = qchunk

        pltpu.sync_copy(cat_v, out_e_hbm.at[pl.ds(base, _BPW)])


@functools.lru_cache(maxsize=None)
def _build_sc_kernel():
    return pl.kernel(
        _sc_body,
        out_type=jax.ShapeDtypeStruct((_K, 2 * _D), jnp.float32),
        mesh=plsc.VectorSubcoreMesh(core_axis_name="c", subcore_axis_name="s"),
        compiler_params=pltpu.CompilerParams(needs_layout_passes=False),
        scratch_types=[
            pltpu.VMEM((16,), jnp.int32),
            pltpu.VMEM((_BPW + 1, _D, 128), jnp.float32),
            pltpu.VMEM((_BPW, 2 * _D), jnp.float32),
            pltpu.SemaphoreType.DMA,
        ],
    )


def _tc_iou_body(sidx_ref, qidx_ref, qblk_ref, *args):
    sblk_refs = args[:_BPW]
    out_ref = args[_BPW]
    g = pl.program_id(0)

    qrow = qblk_ref[pl.ds(qidx_ref[0] % 32, 1), :].astype(jnp.float32)
    rows = [
        sblk_refs[j][pl.ds(sidx_ref[g * _BPW + j] % 32, 1), :]
        .astype(jnp.float32)
        for j in range(_BPW)
    ]
    s_mat = jnp.concatenate(rows + [qrow], axis=0)
    v_mat = jnp.concatenate(
        [qrow, jnp.ones((1, _NU), jnp.float32)], axis=0)
    r = jax.lax.dot_general(
        v_mat, s_mat, (((1,), (1,)), ((), ())),
        preferred_element_type=jnp.float32)
    inter = r[0:1, 0:_BPW]
    s_len = r[1:2, 0:_BPW]
    q_len = r[1:2, _BPW:_BPW + 1]
    union = q_len + s_len - inter
    out_ref[pl.ds(g, 1), :] = jnp.where(
        union > 0, inter / jnp.maximum(union, 1.0), 0.0)


@functools.lru_cache(maxsize=None)
def _build_tc_kernel():
    def sblk_spec(j):
        return pl.BlockSpec(
            (32, _NU), lambda g, sidx, qidx, j=j: (sidx[g * _BPW + j] // 32, 0))

    return pl.pallas_call(
        _tc_iou_body,
        grid_spec=pltpu.PrefetchScalarGridSpec(
            num_scalar_prefetch=2,
            grid=(_K // _BPW,),
            in_specs=[
                pl.BlockSpec((32, _NU), lambda g, sidx, qidx: (qidx[0] // 32, 0)),
            ] + [sblk_spec(j) for j in range(_BPW)],
            out_specs=pl.BlockSpec((_K // _BPW, _BPW),
                                   lambda g, sidx, qidx: (0, 0)),
        ),
        out_shape=jax.ShapeDtypeStruct((_K // _BPW, _BPW), jnp.float32),
    )


def kernel(query, support_set, embed_table, user_sets):
    cat_embeds = _build_sc_kernel()(query, support_set, embed_table.T)
    iou = _build_tc_kernel()(
        support_set, query, *([user_sets] * (_BPW + 1)))
    return cat_embeds, iou.reshape(_K, 1)


# speedup vs baseline: 10.2016x; 1.5788x over previous
import functools

import jax
import jax.numpy as jnp
from jax import lax
from jax.experimental import pallas as pl
from jax.experimental.pallas import tpu as pltpu
from jax.experimental.pallas import tpu_sc as plsc

_NUM_MOVIES = 100000
_D = 64
_NU = 512
_K = 200
_BPW = 8
_NWORK = _K // _BPW
_NS = 16


def _sc_body(q_hbm, idx_hbm, tabT_hbm, out_e_hbm,
             idx_v, slab_v, cat_v, sem):
    wid = lax.axis_index("c") * _NS + lax.axis_index("s")

    @pl.when(wid < _NWORK)
    def _():
        base = wid * _BPW
        pltpu.sync_copy(idx_hbm.at[pl.ds(base, _BPW)], idx_v.at[pl.ds(0, _BPW)])
        pltpu.sync_copy(q_hbm, idx_v.at[pl.ds(_BPW, 1)])
        ivec = idx_v[...]

        copies = []
        for j in range(_BPW + 1):
            m = ivec[j]
            start = pl.multiple_of((m // 128) * 128, 128)
            copies.append(pltpu.make_async_copy(
                tabT_hbm.at[:, pl.ds(start, 128)], slab_v.at[j], sem))
        for c in copies:
            c.start()
        for c in copies:
            c.wait()

        lane = lax.iota(jnp.int32, 16)
        offs = ivec % 128
        qoff = jnp.full((16,), offs[_BPW], jnp.int32)
        for c in range(_D // 16):
            qchunk = plsc.load_gather(
                slab_v, [jnp.full((16,), _BPW, jnp.int32), lane + c * 16,
                         qoff])
            for j in range(_BPW):
                joff = jnp.full((16,), offs[j], jnp.int32)
                cat_v[j, pl.ds(c * 16, 16)] = plsc.load_gather(
                    slab_v, [jnp.full((16,), j, jnp.int32), lane + c * 16,
                             joff])
                cat_v[j, pl.ds(_D + c * 16, 16)] = qchunk

        pltpu.sync_copy(cat_v, out_e_hbm.at[pl.ds(base, _BPW)])


@functools.lru_cache(maxsize=None)
def _build_sc_kernel():
    return pl.kernel(
        _sc_body,
        out_type=jax.ShapeDtypeStruct((_K, 2 * _D), jnp.float32),
        mesh=plsc.VectorSubcoreMesh(core_axis_name="c", subcore_axis_name="s"),
        compiler_params=pltpu.CompilerParams(needs_layout_passes=False),
        scratch_types=[
            pltpu.VMEM((16,), jnp.int32),
            pltpu.VMEM((_BPW + 1, _D, 128), jnp.float32),
            pltpu.VMEM((_BPW, 2 * _D), jnp.float32),
            pltpu.SemaphoreType.DMA,
        ],
    )


def _tc_iou_body(sidx_ref, qidx_ref, qblk_ref, *args):
    sblk_refs = args[:_BPW]
    out_ref = args[_BPW]
    g = pl.program_id(0)

    oh_q = (lax.broadcasted_iota(jnp.int32, (1, 32), 1)
            == qidx_ref[0] % 32).astype(jnp.int8)
    qrow = jax.lax.dot_general(
        oh_q, qblk_ref[...], (((1,), (0,)), ((), ())),
        preferred_element_type=jnp.int32).astype(jnp.int8)
    v_mat = jnp.concatenate(
        [qrow, jnp.ones((1, _NU), jnp.int8)], axis=0)

    blocks = [qblk_ref] + [sblk_refs[j] for j in range(_BPW)]
    parts = [
        jax.lax.dot_general(
            v_mat, blk[...], (((1,), (1,)), ((), ())),
            preferred_element_type=jnp.int32)
        for blk in blocks
    ]
    p_all = jnp.concatenate(parts, axis=1).astype(jnp.float32)

    cols = [jnp.full((1, 1), qidx_ref[0] % 32, jnp.int32)]
    for j in range(_BPW):
        cols.append(jnp.full((1, 1),
                             32 * (j + 1) + sidx_ref[g * _BPW + j] % 32,
                             jnp.int32))
    colv = jnp.concatenate(cols, axis=0)
    oh = (lax.broadcasted_iota(jnp.int32, (_BPW + 1, 32 * (_BPW + 1)), 1)
          == colv).astype(jnp.float32)
    r = jax.lax.dot_general(
        p_all, oh, (((1,), (1,)), ((), ())),
        preferred_element_type=jnp.float32)
    inter = r[0:1, 1:]
    s_len = r[1:2, 1:]
    q_len = r[1:2, 0:1]
    union = q_len + s_len - inter
    out_ref[pl.ds(g, 1), :] = jnp.where(
        union > 0, inter / jnp.maximum(union, 1.0), 0.0)


@functools.lru_cache(maxsize=None)
def _build_tc_kernel():
    def sblk_spec(j):
        return pl.BlockSpec(
            (32, _NU), lambda g, sidx, qidx, j=j: (sidx[g * _BPW + j] // 32, 0))

    return pl.pallas_call(
        _tc_iou_body,
        grid_spec=pltpu.PrefetchScalarGridSpec(
            num_scalar_prefetch=2,
            grid=(_K // _BPW,),
            in_specs=[
                pl.BlockSpec((32, _NU), lambda g, sidx, qidx: (qidx[0] // 32, 0)),
            ] + [sblk_spec(j) for j in range(_BPW)],
            out_specs=pl.BlockSpec((_K // _BPW, _BPW),
                                   lambda g, sidx, qidx: (0, 0)),
        ),
        out_shape=jax.ShapeDtypeStruct((_K // _BPW, _BPW), jnp.float32),
    )


def kernel(query, support_set, embed_table, user_sets):
    cat_embeds = _build_sc_kernel()(query, support_set, embed_table.T)
    u8 = user_sets.astype(jnp.int8)
    iou = _build_tc_kernel()(
        support_set, query, *([u8] * (_BPW + 1)))
    return cat_embeds, iou.reshape(_K, 1)


# speedup vs baseline: 11.3056x; 1.1082x over previous
import functools

import jax
import jax.numpy as jnp
from jax import lax
from jax.experimental import pallas as pl
from jax.experimental.pallas import tpu as pltpu
from jax.experimental.pallas import tpu_sc as plsc

_NUM_MOVIES = 100000
_D = 64
_NU = 512
_K = 200
_BPW = 8
_NWORK = _K // _BPW
_NS = 16
_TCB = 16
_KPAD = 208


def _sc_body(q_hbm, idx_hbm, tabT_hbm, out_e_hbm,
             idx_v, slab_v, cat_v, sem):
    wid = lax.axis_index("c") * _NS + lax.axis_index("s")

    @pl.when(wid < _NWORK)
    def _():
        base = wid * _BPW
        pltpu.sync_copy(idx_hbm.at[pl.ds(base, _BPW)], idx_v.at[pl.ds(0, _BPW)])
        pltpu.sync_copy(q_hbm, idx_v.at[pl.ds(_BPW, 1)])
        ivec = idx_v[...]

        copies = []
        for j in range(_BPW + 1):
            m = ivec[j]
            start = pl.multiple_of((m // 128) * 128, 128)
            copies.append(pltpu.make_async_copy(
                tabT_hbm.at[:, pl.ds(start, 128)], slab_v.at[j], sem))
        for c in copies:
            c.start()
        for c in copies:
            c.wait()

        lane = lax.iota(jnp.int32, 16)
        offs = ivec % 128
        qoff = jnp.full((16,), offs[_BPW], jnp.int32)
        for c in range(_D // 16):
            qchunk = plsc.load_gather(
                slab_v, [jnp.full((16,), _BPW, jnp.int32), lane + c * 16,
                         qoff])
            for j in range(_BPW):
                joff = jnp.full((16,), offs[j], jnp.int32)
                cat_v[j, pl.ds(c * 16, 16)] = plsc.load_gather(
                    slab_v, [jnp.full((16,), j, jnp.int32), lane + c * 16,
                             joff])
                cat_v[j, pl.ds(_D + c * 16, 16)] = qchunk

        pltpu.sync_copy(cat_v, out_e_hbm.at[pl.ds(base, _BPW)])


@functools.lru_cache(maxsize=None)
def _build_sc_kernel():
    return pl.kernel(
        _sc_body,
        out_type=jax.ShapeDtypeStruct((_K, 2 * _D), jnp.float32),
        mesh=plsc.VectorSubcoreMesh(core_axis_name="c", subcore_axis_name="s"),
        compiler_params=pltpu.CompilerParams(needs_layout_passes=False),
        scratch_types=[
            pltpu.VMEM((16,), jnp.int32),
            pltpu.VMEM((_BPW + 1, _D, 128), jnp.float32),
            pltpu.VMEM((_BPW, 2 * _D), jnp.float32),
            pltpu.SemaphoreType.DMA,
        ],
    )


def _tc_iou_body(sidx_ref, qidx_ref, qblk_ref, *args):
    sblk_refs = args[:_TCB]
    out_ref = args[_TCB]
    g = pl.program_id(0)
    nb = _TCB + 1

    oh_q = (lax.broadcasted_iota(jnp.int32, (1, 32), 1)
            == qidx_ref[0] % 32).astype(jnp.int8)
    qrow = jax.lax.dot_general(
        oh_q, qblk_ref[...], (((1,), (0,)), ((), ())),
        preferred_element_type=jnp.int32).astype(jnp.int8)
    v_mat = jnp.concatenate(
        [qrow, jnp.ones((1, _NU), jnp.int8)], axis=0)

    s_all = jnp.concatenate(
        [qblk_ref[...]] + [sblk_refs[j][...] for j in range(_TCB)],
        axis=0)
    p_all = jax.lax.dot_general(
        v_mat, s_all, (((1,), (1,)), ((), ())),
        preferred_element_type=jnp.int32).astype(jnp.float32)

    cols = [jnp.full((1, 1), qidx_ref[0] % 32, jnp.int32)]
    for j in range(_TCB):
        cols.append(jnp.full((1, 1),
                             32 * (j + 1) + sidx_ref[g * _TCB + j] % 32,
                             jnp.int32))
    colv = jnp.concatenate(cols, axis=0)
    oh = (lax.broadcasted_iota(jnp.int32, (nb, 32 * nb), 1)
          == colv).astype(jnp.float32)
    r = jax.lax.dot_general(
        p_all, oh, (((1,), (1,)), ((), ())),
        preferred_element_type=jnp.float32)
    inter = r[0:1, 1:]
    s_len = r[1:2, 1:]
    q_len = r[1:2, 0:1]
    union = q_len + s_len - inter
    out_ref[pl.ds(g, 1), :] = jnp.where(
        union > 0, inter / jnp.maximum(union, 1.0), 0.0)


@functools.lru_cache(maxsize=None)
def _build_tc_kernel():
    def sblk_spec(j):
        return pl.BlockSpec(
            (32, _NU), lambda g, sidx, qidx, j=j: (sidx[g * _TCB + j] // 32, 0))

    return pl.pallas_call(
        _tc_iou_body,
        grid_spec=pltpu.PrefetchScalarGridSpec(
            num_scalar_prefetch=2,
            grid=(_KPAD // _TCB,),
            in_specs=[
                pl.BlockSpec((32, _NU), lambda g, sidx, qidx: (qidx[0] // 32, 0)),
            ] + [sblk_spec(j) for j in range(_TCB)],
            out_specs=pl.BlockSpec((_KPAD // _TCB, _TCB),
                                   lambda g, sidx, qidx: (0, 0)),
        ),
        out_shape=jax.ShapeDtypeStruct((_KPAD // _TCB, _TCB), jnp.float32),
    )


def kernel(query, support_set, embed_table, user_sets):
    cat_embeds = _build_sc_kernel()(query, support_set, embed_table.T)
    u8 = user_sets.astype(jnp.int8)
    sup_pad = jnp.concatenate(
        [support_set, jnp.zeros((_KPAD - _K,), jnp.int32)])
    iou = _build_tc_kernel()(
        sup_pad, query, *([u8] * (_TCB + 1)))
    return cat_embeds, iou.reshape(_KPAD, 1)[:_K]


# speedup vs baseline: 12.7456x; 1.1274x over previous
import functools

import jax
import jax.numpy as jnp
from jax import lax
from jax.experimental import pallas as pl
from jax.experimental.pallas import tpu as pltpu
from jax.experimental.pallas import tpu_sc as plsc

_NUM_MOVIES = 100000
_D = 64
_NU = 512
_K = 200
_BPW = 8
_NWORK = _K // _BPW
_NS = 16
_TCB = 20
_KPAD = _K


def _sc_body(q_hbm, idx_hbm, tabT_hbm, out_e_hbm,
             idx_v, slab_v, cat_v, sem):
    wid = lax.axis_index("c") * _NS + lax.axis_index("s")

    @pl.when(wid < _NWORK)
    def _():
        base = wid * _BPW
        pltpu.sync_copy(idx_hbm.at[pl.ds(base, _BPW)], idx_v.at[pl.ds(0, _BPW)])
        pltpu.sync_copy(q_hbm, idx_v.at[pl.ds(_BPW, 1)])
        ivec = idx_v[...]

        copies = []
        for j in range(_BPW + 1):
            m = ivec[j]
            start = pl.multiple_of((m // 128) * 128, 128)
            copies.append(pltpu.make_async_copy(
                tabT_hbm.at[:, pl.ds(start, 128)], slab_v.at[j], sem))
        for c in copies:
            c.start()
        for c in copies:
            c.wait()

        lane = lax.iota(jnp.int32, 16)
        offs = ivec % 128
        qoff = jnp.full((16,), offs[_BPW], jnp.int32)
        for c in range(_D // 16):
            qchunk = plsc.load_gather(
                slab_v, [jnp.full((16,), _BPW, jnp.int32), lane + c * 16,
                         qoff])
            for j in range(_BPW):
                joff = jnp.full((16,), offs[j], jnp.int32)
                cat_v[j, pl.ds(c * 16, 16)] = plsc.load_gather(
                    slab_v, [jnp.full((16,), j, jnp.int32), lane + c * 16,
                             joff])
                cat_v[j, pl.ds(_D + c * 16, 16)] = qchunk

        pltpu.sync_copy(cat_v, out_e_hbm.at[pl.ds(base, _BPW)])


@functools.lru_cache(maxsize=None)
def _build_sc_kernel():
    return pl.kernel(
        _sc_body,
        out_type=jax.ShapeDtypeStruct((_K, 2 * _D), jnp.float32),
        mesh=plsc.VectorSubcoreMesh(core_axis_name="c", subcore_axis_name="s"),
        compiler_params=pltpu.CompilerParams(needs_layout_passes=False),
        scratch_types=[
            pltpu.VMEM((16,), jnp.int32),
            pltpu.VMEM((_BPW + 1, _D, 128), jnp.float32),
            pltpu.VMEM((_BPW, 2 * _D), jnp.float32),
            pltpu.SemaphoreType.DMA,
        ],
    )


def _tc_iou_body(sidx_ref, qidx_ref, qblk_ref, *args):
    sblk_refs = args[:_TCB]
    out_ref = args[_TCB]
    g = pl.program_id(0)
    nb = _TCB + 1

    oh_q = (lax.broadcasted_iota(jnp.int32, (1, 32), 1)
            == qidx_ref[0] % 32).astype(jnp.int8)
    qrow = jax.lax.dot_general(
        oh_q, qblk_ref[...].astype(jnp.int8), (((1,), (0,)), ((), ())),
        preferred_element_type=jnp.int32).astype(jnp.int8)
    v_mat = jnp.concatenate(
        [qrow, jnp.ones((1, _NU), jnp.int8)], axis=0)

    s_all = jnp.concatenate(
        [qblk_ref[...].astype(jnp.int8)]
        + [sblk_refs[j][...].astype(jnp.int8) for j in range(_TCB)],
        axis=0)
    p_all = jax.lax.dot_general(
        v_mat, s_all, (((1,), (1,)), ((), ())),
        preferred_element_type=jnp.int32).astype(jnp.float32)

    cols = [jnp.full((1, 1), qidx_ref[0] % 32, jnp.int32)]
    for j in range(_TCB):
        cols.append(jnp.full((1, 1),
                             32 * (j + 1) + sidx_ref[g * _TCB + j] % 32,
                             jnp.int32))
    colv = jnp.concatenate(cols, axis=0)
    oh = (lax.broadcasted_iota(jnp.int32, (nb, 32 * nb), 1)
          == colv).astype(jnp.float32)
    r = jax.lax.dot_general(
        p_all, oh, (((1,), (1,)), ((), ())),
        preferred_element_type=jnp.float32)
    inter = r[0:1, 1:]
    s_len = r[1:2, 1:]
    q_len = r[1:2, 0:1]
    union = q_len + s_len - inter
    out_ref[pl.ds(g, 1), :] = jnp.where(
        union > 0, inter / jnp.maximum(union, 1.0), 0.0)


@functools.lru_cache(maxsize=None)
def _build_tc_kernel():
    def sblk_spec(j):
        return pl.BlockSpec(
            (32, _NU), lambda g, sidx, qidx, j=j: (sidx[g * _TCB + j] // 32, 0))

    return pl.pallas_call(
        _tc_iou_body,
        grid_spec=pltpu.PrefetchScalarGridSpec(
            num_scalar_prefetch=2,
            grid=(_KPAD // _TCB,),
            in_specs=[
                pl.BlockSpec((32, _NU), lambda g, sidx, qidx: (qidx[0] // 32, 0)),
            ] + [sblk_spec(j) for j in range(_TCB)],
            out_specs=pl.BlockSpec((_KPAD // _TCB, _TCB),
                                   lambda g, sidx, qidx: (0, 0)),
        ),
        out_shape=jax.ShapeDtypeStruct((_KPAD // _TCB, _TCB), jnp.float32),
    )


def kernel(query, support_set, embed_table, user_sets):
    cat_embeds = _build_sc_kernel()(query, support_set, embed_table.T)
    u4 = user_sets.astype(jnp.int4)
    iou = _build_tc_kernel()(
        support_set, query, *([u4] * (_TCB + 1)))
    return cat_embeds, iou.reshape(_K, 1)
